# baseline (speedup 1.0000x reference)
import functools

import jax
import jax.numpy as jnp
from jax import lax
from jax.experimental import pallas as pl
from jax.experimental.pallas import tpu as pltpu
from jax.experimental.pallas import tpu_sc as plsc

N = 10000
KNN = 16
NP = 10240
RB = 256
CB = 2048
NCH = NP // CB
KPAD = 128

def _big():
    return jnp.int32(2 ** 30)


NCLS = 8
CW = CB // NCLS

def _batcher_pairs(n):
    pairs = []
    p = 1
    while p < n:
        k = p
        while k >= 1:
            for j in range(k % p, n - k, 2 * k):
                for i in range(0, k):
                    if (i + j) // (p * 2) == (i + j + k) // (p * 2):
                        pairs.append((i + j, i + j + k))
            k //= 2
        p *= 2
    return pairs


_SORT8 = _batcher_pairs(NCLS)


def _knn_body(xp_ref, xTp_ref, out_ref):
    rb = pl.program_id(0)
    xr = xp_ref[...]
    sqr = jnp.sum(xr * xr, axis=1, keepdims=True)
    row_id = rb * RB + lax.broadcasted_iota(jnp.int32, (RB, CW), 0)
    inf = jnp.float32(jnp.inf)
    pos = lax.broadcasted_iota(jnp.int32, (RB, CW), 1)
    run_v = run_i = None
    for c in range(NCH):
        xc = xTp_ref[:, c * CB:(c + 1) * CB]
        dot = lax.dot_general(xr, xc, (((1,), (0,)), ((), ())),
                              preferred_element_type=jnp.float32)
        sqc = jnp.sum(xc * xc, axis=0, keepdims=True)
        d2 = sqr - 2.0 * dot + sqc
        V, I = [], []
        for j in range(NCLS):
            col_j = c * CB + NCLS * pos + j
            vj = d2[:, j * CW:(j + 1) * CW]
            vj = jnp.where((col_j == row_id) | (col_j >= N), inf, vj)
            V.append(vj)
            I.append(col_j)
        for (i, j) in _SORT8:
            swap = V[j] < V[i]
            lo_v = jnp.where(swap, V[j], V[i])
            hi_v = jnp.where(swap, V[i], V[j])
            lo_i = jnp.where(swap, I[j], I[i])
            hi_i = jnp.where(swap, I[i], I[j])
            V[i], V[j], I[i], I[j] = lo_v, hi_v, lo_i, hi_i
        vl, il = [], []
        for _ in range(KNN):
            m = jnp.min(V[0], axis=1, keepdims=True)
            ap = jnp.min(jnp.where(V[0] == m, pos, _big()), axis=1,
                         keepdims=True)
            iv = jnp.min(jnp.where(pos == ap, I[0], _big()), axis=1,
                         keepdims=True)
            vl.append(m)
            il.append(iv)
            hit = pos == ap
            for k in range(NCLS - 1):
                V[k] = jnp.where(hit, V[k + 1], V[k])
                I[k] = jnp.where(hit, I[k + 1], I[k])
            V[NCLS - 1] = jnp.where(hit, inf, V[NCLS - 1])
        nv = jnp.concatenate(vl, axis=1)
        ni = jnp.concatenate(il, axis=1)
        if run_v is None:
            run_v, run_i = nv, ni
        else:
            cv = jnp.concatenate([run_v, nv], axis=1)
            ci = jnp.concatenate([run_i, ni], axis=1)
            mpos = lax.broadcasted_iota(jnp.int32, (RB, 2 * KNN), 1)
            vl, il = [], []
            for _ in range(KNN):
                m = jnp.min(cv, axis=1, keepdims=True)
                ap = jnp.min(jnp.where(cv == m, mpos, _big()), axis=1,
                             keepdims=True)
                iv = jnp.min(jnp.where(mpos == ap, ci, _big()), axis=1,
                             keepdims=True)
                vl.append(m)
                il.append(iv)
                cv = jnp.where(mpos == ap, inf, cv)
            run_v = jnp.concatenate(vl, axis=1)
            run_i = jnp.concatenate(il, axis=1)
    pad = jnp.zeros((RB, 128 - KNN), jnp.int32)
    out_ref[...] = jnp.concatenate([run_i, pad], axis=1)


def _knn_idx(xp, xT):
    idx128 = pl.pallas_call(
        _knn_body,
        grid=(NP // RB,),
        in_specs=[pl.BlockSpec((RB, KPAD), lambda r: (r, 0)),
                  pl.BlockSpec((KPAD, NP), lambda r: (0, 0))],
        out_specs=pl.BlockSpec((RB, 128), lambda r: (r, 0)),
        out_shape=jax.ShapeDtypeStruct((NP, 128), jnp.int32),
    )(xp, xT)
    return idx128[:, :KNN]


def _mm_body(h_ref, w_ref, o_ref):
    o_ref[...] = lax.dot_general(h_ref[...], w_ref[...],
                                 (((1,), (0,)), ((), ())),
                                 precision=lax.Precision.HIGHEST,
                                 preferred_element_type=jnp.float32)


def _matmul(h_aug, w_cat):
    kp = h_aug.shape[1]
    d2 = w_cat.shape[1]
    mb = 1024
    return pl.pallas_call(
        _mm_body,
        grid=(NP // mb,),
        in_specs=[pl.BlockSpec((mb, kp), lambda r: (r, 0)),
                  pl.BlockSpec((kp, d2), lambda r: (0, 0))],
        out_specs=pl.BlockSpec((mb, d2), lambda r: (r, 0)),
        out_shape=jax.ShapeDtypeStruct((NP, d2), jnp.float32),
    )(h_aug, w_cat)


_NB = 8


def _make_gathermax(d):
    nw = 32
    npw = NP // nw
    nch = npw // _NB
    mesh = plsc.VectorSubcoreMesh(core_axis_name="c", subcore_axis_name="s")

    @functools.partial(
        pl.kernel,
        out_type=jax.ShapeDtypeStruct((NP, d), jnp.float32),
        mesh=mesh,
        scratch_types=[
            pltpu.VMEM((_NB * KNN,), jnp.int32),
            pltpu.VMEM((_NB * KNN, d), jnp.float32),
            pltpu.VMEM((_NB, d), jnp.float32),
            pltpu.VMEM((_NB, d), jnp.float32),
            pltpu.SemaphoreType.DMA,
        ],
    )
    def gathermax(idx_hbm, a_hbm, c_hbm, out_hbm, idx_v, rows_v, a_v, o_v,
                  sem):
        wid = lax.axis_index("s") * 2 + lax.axis_index("c")
        node0 = wid * npw

        def chunk(t, carry):
            nb = node0 + t * _NB
            pltpu.sync_copy(idx_hbm.at[pl.ds(nb * KNN, _NB * KNN)], idx_v)
            pltpu.async_copy(c_hbm.at[idx_v], rows_v, sem).wait()
            pltpu.sync_copy(a_hbm.at[pl.ds(nb, _NB)], a_v)

            def node(i, c2):
                for v in range(d // 16):
                    sl = pl.ds(v * 16, 16)
                    acc = rows_v[i * KNN, sl]
                    for j in range(1, KNN):
                        acc = jnp.maximum(acc, rows_v[i * KNN + j, sl])
                    o_v[i, sl] = jnp.maximum(acc + a_v[i, sl], 0.0)
                return c2

            lax.fori_loop(0, _NB, node, 0)
            pltpu.sync_copy(o_v, out_hbm.at[pl.ds(nb, _NB)])
            return carry

        lax.fori_loop(0, nch, chunk, 0)

    return gathermax


def _layer(h, W, b, d_in, d_out, idx_flat):
    wa, wb = W[:d_in], W[d_in:]
    kp = -(-(d_in + 1) // 8) * 8
    dg = max(d_out, 128)
    w_cat = jnp.zeros((kp, 2 * dg), jnp.float32)
    w_cat = w_cat.at[:d_in, :d_out].set(wa - wb)
    w_cat = w_cat.at[d_in, :d_out].set(b)
    w_cat = w_cat.at[:d_in, dg:dg + d_out].set(wb)
    h_aug = jnp.zeros((NP, kp), jnp.float32)
    h_aug = h_aug.at[:, :d_in].set(h)
    h_aug = h_aug.at[:, d_in].set(1.0)
    ac = _matmul(h_aug, w_cat)
    a = ac[:, :dg]
    c = ac[:, dg:]
    out = _make_gathermax(dg)(idx_flat, a, c)
    return out[:, :d_out]


def _perm_cols():
    import numpy as np
    p = np.arange(NP)
    ch, q = p // CB, p % CB
    j, cc = q // CW, q % CW
    return ch * CB + NCLS * cc + j


_PERM = _perm_cols()


def kernel(x, W0, b0, W1, b1, W2, b2):
    xp = jnp.zeros((NP, KPAD), jnp.float32).at[:N, :3].set(x)
    idx = _knn_idx(xp, xp.T[:, _PERM])
    idx_flat = idx.reshape(-1)
    h = _layer(xp[:, :3], W0, b0, 3, 64, idx_flat)
    h = _layer(h, W1, b1, 64, 128, idx_flat)
    h = _layer(h, W2, b2, 128, 256, idx_flat)
    return h[:N]


# speedup vs baseline: 1.2229x; 1.2229x over previous
import functools

import jax
import jax.numpy as jnp
from jax import lax
from jax.experimental import pallas as pl
from jax.experimental.pallas import tpu as pltpu
from jax.experimental.pallas import tpu_sc as plsc

N = 10000
KNN = 16
NP = 10240
RB = 256
CB = 2048
NCH = NP // CB
KPAD = 128

def _big():
    return jnp.int32(2 ** 30)


def _knn_body(xp_ref, xT_ref, out_ref):
    rb = pl.program_id(0)
    xr = xp_ref[...]
    sqr = jnp.sum(xr * xr, axis=1, keepdims=True)
    row_id = rb * RB + lax.broadcasted_iota(jnp.int32, (RB, CB), 0)
    inf = jnp.float32(jnp.inf)
    run_v = run_i = None
    for c in range(NCH):
        xc = xT_ref[:, c * CB:(c + 1) * CB]
        dot = lax.dot_general(xr, xc, (((1,), (0,)), ((), ())),
                              preferred_element_type=jnp.float32)
        sqc = jnp.sum(xc * xc, axis=0, keepdims=True)
        col = c * CB + lax.broadcasted_iota(jnp.int32, (RB, CB), 1)
        d2 = sqr - 2.0 * dot + sqc
        d2 = jnp.where((col == row_id) | (col >= N), inf, d2)
        vl, il = [], []
        for _ in range(KNN):
            m = jnp.min(d2, axis=1, keepdims=True)
            am = jnp.min(jnp.where(d2 == m, col, _big()), axis=1,
                         keepdims=True)
            vl.append(m)
            il.append(am)
            d2 = jnp.where(col == am, inf, d2)
        nv = jnp.concatenate(vl, axis=1)
        ni = jnp.concatenate(il, axis=1)
        if run_v is None:
            run_v, run_i = nv, ni
        else:
            cv = jnp.concatenate([run_v, nv], axis=1)
            ci = jnp.concatenate([run_i, ni], axis=1)
            mpos = lax.broadcasted_iota(jnp.int32, (RB, 2 * KNN), 1)
            vl, il = [], []
            for _ in range(KNN):
                m = jnp.min(cv, axis=1, keepdims=True)
                ap = jnp.min(jnp.where(cv == m, mpos, _big()), axis=1,
                             keepdims=True)
                iv = jnp.min(jnp.where(mpos == ap, ci, _big()), axis=1,
                             keepdims=True)
                vl.append(m)
                il.append(iv)
                cv = jnp.where(mpos == ap, inf, cv)
            run_v = jnp.concatenate(vl, axis=1)
            run_i = jnp.concatenate(il, axis=1)
    pad = jnp.zeros((RB, 128 - KNN), jnp.int32)
    out_ref[...] = jnp.concatenate([run_i, pad], axis=1)


def _knn_idx(xp, xT):
    idx128 = pl.pallas_call(
        _knn_body,
        grid=(NP // RB,),
        in_specs=[pl.BlockSpec((RB, KPAD), lambda r: (r, 0)),
                  pl.BlockSpec((KPAD, NP), lambda r: (0, 0))],
        out_specs=pl.BlockSpec((RB, 128), lambda r: (r, 0)),
        out_shape=jax.ShapeDtypeStruct((NP, 128), jnp.int32),
    )(xp, xT)
    return idx128[:, :KNN]


def _mm_body(h_ref, w_ref, o_ref):
    o_ref[...] = lax.dot_general(h_ref[...], w_ref[...],
                                 (((1,), (0,)), ((), ())),
                                 precision=lax.Precision.HIGHEST,
                                 preferred_element_type=jnp.float32)


def _matmul(h_aug, w_cat):
    kp = h_aug.shape[1]
    d2 = w_cat.shape[1]
    mb = 1024
    return pl.pallas_call(
        _mm_body,
        grid=(NP // mb,),
        in_specs=[pl.BlockSpec((mb, kp), lambda r: (r, 0)),
                  pl.BlockSpec((kp, d2), lambda r: (0, 0))],
        out_specs=pl.BlockSpec((mb, d2), lambda r: (r, 0)),
        out_shape=jax.ShapeDtypeStruct((NP, d2), jnp.float32),
    )(h_aug, w_cat)


_NB = 8


def _make_gathermax(d):
    nw = 32
    npw = NP // nw
    nch = npw // _NB
    mesh = plsc.VectorSubcoreMesh(core_axis_name="c", subcore_axis_name="s")

    @functools.partial(
        pl.kernel,
        out_type=jax.ShapeDtypeStruct((NP, d), jnp.float32),
        mesh=mesh,
        scratch_types=[
            pltpu.VMEM((_NB * KNN,), jnp.int32),
            pltpu.VMEM((_NB * KNN,), jnp.int32),
            pltpu.VMEM((_NB * KNN, d), jnp.float32),
            pltpu.VMEM((_NB * KNN, d), jnp.float32),
            pltpu.VMEM((_NB, d), jnp.float32),
            pltpu.VMEM((_NB, d), jnp.float32),
            pltpu.VMEM((_NB, d), jnp.float32),
            pltpu.SemaphoreType.DMA,
            pltpu.SemaphoreType.DMA,
            pltpu.SemaphoreType.DMA,
            pltpu.SemaphoreType.DMA,
        ],
    )
    def gathermax(idx_hbm, a_hbm, c_hbm, out_hbm, i0, i1, r0, r1, a0, a1,
                  o_v, g0, g1, s0, s1):
        wid = lax.axis_index("s") * 2 + lax.axis_index("c")
        node0 = wid * npw
        iv, rv, av = [i0, i1], [r0, r1], [a0, a1]
        gsem, asem = [g0, g1], [s0, s1]

        def start(b, t):
            nb = node0 + t * _NB
            pltpu.sync_copy(idx_hbm.at[pl.ds(nb * KNN, _NB * KNN)], iv[b])
            pltpu.async_copy(c_hbm.at[iv[b]], rv[b], gsem[b])
            pltpu.async_copy(a_hbm.at[pl.ds(nb, _NB)], av[b], asem[b])

        for b in range(2):
            start(b, b)

        def pair(u, carry):
            for b in range(2):
                t = 2 * u + b
                nb = node0 + t * _NB
                pltpu.make_async_copy(
                    c_hbm.at[pl.ds(0, _NB * KNN)], rv[b], gsem[b]).wait()
                pltpu.make_async_copy(
                    a_hbm.at[pl.ds(0, _NB)], av[b], asem[b]).wait()

                def node(i, c2, _b=b):
                    for v in range(d // 16):
                        sl = pl.ds(v * 16, 16)
                        acc = rv[_b][i * KNN, sl]
                        for j in range(1, KNN):
                            acc = jnp.maximum(acc, rv[_b][i * KNN + j, sl])
                        o_v[i, sl] = jnp.maximum(acc + av[_b][i, sl], 0.0)
                    return c2

                lax.fori_loop(0, _NB, node, 0)
                pltpu.sync_copy(o_v, out_hbm.at[pl.ds(nb, _NB)])

                @pl.when(t + 2 < nch)
                def _():
                    start(b, t + 2)
            return carry

        lax.fori_loop(0, nch // 2, pair, 0)

    return gathermax


def _layer(h, W, b, d_in, d_out, idx_flat):
    wa, wb = W[:d_in], W[d_in:]
    kp = -(-(d_in + 1) // 8) * 8
    dg = max(d_out, 128)
    w_cat = jnp.zeros((kp, 2 * dg), jnp.float32)
    w_cat = w_cat.at[:d_in, :d_out].set(wa - wb)
    w_cat = w_cat.at[d_in, :d_out].set(b)
    w_cat = w_cat.at[:d_in, dg:dg + d_out].set(wb)
    h_aug = jnp.zeros((NP, kp), jnp.float32)
    h_aug = h_aug.at[:, :d_in].set(h)
    h_aug = h_aug.at[:, d_in].set(1.0)
    ac = _matmul(h_aug, w_cat)
    a = ac[:, :dg]
    c = ac[:, dg:]
    out = _make_gathermax(dg)(idx_flat, a, c)
    return out[:, :d_out]


def kernel(x, W0, b0, W1, b1, W2, b2):
    xp = jnp.zeros((NP, KPAD), jnp.float32).at[:N, :3].set(x)
    idx = _knn_idx(xp, xp.T)
    idx_flat = idx.reshape(-1)
    h = _layer(xp[:, :3], W0, b0, 3, 64, idx_flat)
    h = _layer(h, W1, b1, 64, 128, idx_flat)
    h = _layer(h, W2, b2, 128, 256, idx_flat)
    return h[:N]


# speedup vs baseline: 1.3922x; 1.1385x over previous
import functools

import jax
import jax.numpy as jnp
from jax import lax
from jax.experimental import pallas as pl
from jax.experimental.pallas import tpu as pltpu
from jax.experimental.pallas import tpu_sc as plsc

N = 10000
KNN = 16
NP = 10240
RB = 512
CB = 2048
NCH = NP // CB
KPAD = 128

def _big():
    return jnp.int32(2 ** 30)


def _knn_body(xp_ref, xT_ref, out_ref):
    rb = pl.program_id(0)
    xr = xp_ref[...]
    sqr = jnp.sum(xr * xr, axis=1, keepdims=True)
    row_id = rb * RB + lax.broadcasted_iota(jnp.int32, (RB, CB), 0)
    inf = jnp.float32(jnp.inf)
    run_v = run_i = None
    for c in range(NCH):
        xc = xT_ref[:, c * CB:(c + 1) * CB]
        dot = lax.dot_general(xr, xc, (((1,), (0,)), ((), ())),
                              preferred_element_type=jnp.float32)
        sqc = jnp.sum(xc * xc, axis=0, keepdims=True)
        col = c * CB + lax.broadcasted_iota(jnp.int32, (RB, CB), 1)
        d2 = sqr - 2.0 * dot + sqc
        d2 = jnp.where((col == row_id) | (col >= N), inf, d2)
        vl, il = [], []
        for _ in range(KNN):
            m = jnp.min(d2, axis=1, keepdims=True)
            eq = d2 == m
            am = jnp.min(jnp.where(eq, col, _big()), axis=1, keepdims=True)
            vl.append(m)
            il.append(am)
            d2 = jnp.where(eq, inf, d2)
        nv = jnp.concatenate(vl, axis=1)
        ni = jnp.concatenate(il, axis=1)
        if run_v is None:
            run_v, run_i = nv, ni
        else:
            cv = jnp.concatenate([run_v, nv], axis=1)
            ci = jnp.concatenate([run_i, ni], axis=1)
            mpos = lax.broadcasted_iota(jnp.int32, (RB, 2 * KNN), 1)
            vl, il = [], []
            for _ in range(KNN):
                m = jnp.min(cv, axis=1, keepdims=True)
                ap = jnp.min(jnp.where(cv == m, mpos, _big()), axis=1,
                             keepdims=True)
                iv = jnp.min(jnp.where(mpos == ap, ci, _big()), axis=1,
                             keepdims=True)
                vl.append(m)
                il.append(iv)
                cv = jnp.where(mpos == ap, inf, cv)
            run_v = jnp.concatenate(vl, axis=1)
            run_i = jnp.concatenate(il, axis=1)
    pad = jnp.zeros((RB, 128 - KNN), jnp.int32)
    out_ref[...] = jnp.concatenate([run_i, pad], axis=1)


def _knn_idx(xp, xT):
    idx128 = pl.pallas_call(
        _knn_body,
        grid=(NP // RB,),
        in_specs=[pl.BlockSpec((RB, KPAD), lambda r: (r, 0)),
                  pl.BlockSpec((KPAD, NP), lambda r: (0, 0))],
        out_specs=pl.BlockSpec((RB, 128), lambda r: (r, 0)),
        out_shape=jax.ShapeDtypeStruct((NP, 128), jnp.int32),
    )(xp, xT)
    return idx128[:, :KNN]


def _mm_body(h_ref, w_ref, o_ref):
    o_ref[...] = lax.dot_general(h_ref[...], w_ref[...],
                                 (((1,), (0,)), ((), ())),
                                 precision=lax.Precision.HIGHEST,
                                 preferred_element_type=jnp.float32)


def _matmul(h_aug, w_cat):
    kp = h_aug.shape[1]
    d2 = w_cat.shape[1]
    mb = 1024
    return pl.pallas_call(
        _mm_body,
        grid=(NP // mb,),
        in_specs=[pl.BlockSpec((mb, kp), lambda r: (r, 0)),
                  pl.BlockSpec((kp, d2), lambda r: (0, 0))],
        out_specs=pl.BlockSpec((mb, d2), lambda r: (r, 0)),
        out_shape=jax.ShapeDtypeStruct((NP, d2), jnp.float32),
    )(h_aug, w_cat)


_NB = 8


def _make_gathermax(d):
    nw = 32
    npw = NP // nw
    nch = npw // _NB
    mesh = plsc.VectorSubcoreMesh(core_axis_name="c", subcore_axis_name="s")

    @functools.partial(
        pl.kernel,
        out_type=jax.ShapeDtypeStruct((NP, d), jnp.float32),
        mesh=mesh,
        scratch_types=[
            pltpu.VMEM((_NB * KNN,), jnp.int32),
            pltpu.VMEM((_NB * KNN,), jnp.int32),
            pltpu.VMEM((_NB * KNN, d), jnp.float32),
            pltpu.VMEM((_NB * KNN, d), jnp.float32),
            pltpu.VMEM((_NB, d), jnp.float32),
            pltpu.VMEM((_NB, d), jnp.float32),
            pltpu.VMEM((_NB, d), jnp.float32),
            pltpu.SemaphoreType.DMA,
            pltpu.SemaphoreType.DMA,
            pltpu.SemaphoreType.DMA,
            pltpu.SemaphoreType.DMA,
        ],
    )
    def gathermax(idx_hbm, a_hbm, c_hbm, out_hbm, i0, i1, r0, r1, a0, a1,
                  o_v, g0, g1, s0, s1):
        wid = lax.axis_index("s") * 2 + lax.axis_index("c")
        node0 = wid * npw
        iv, rv, av = [i0, i1], [r0, r1], [a0, a1]
        gsem, asem = [g0, g1], [s0, s1]

        def start(b, t):
            nb = node0 + t * _NB
            pltpu.sync_copy(idx_hbm.at[pl.ds(nb * KNN, _NB * KNN)], iv[b])
            pltpu.async_copy(c_hbm.at[iv[b]], rv[b], gsem[b])
            pltpu.async_copy(a_hbm.at[pl.ds(nb, _NB)], av[b], asem[b])

        for b in range(2):
            start(b, b)

        def pair(u, carry):
            for b in range(2):
                t = 2 * u + b
                nb = node0 + t * _NB
                pltpu.make_async_copy(
                    c_hbm.at[pl.ds(0, _NB * KNN)], rv[b], gsem[b]).wait()
                pltpu.make_async_copy(
                    a_hbm.at[pl.ds(0, _NB)], av[b], asem[b]).wait()

                def node(i, c2, _b=b):
                    for v in range(d // 16):
                        sl = pl.ds(v * 16, 16)
                        acc = rv[_b][i * KNN, sl]
                        for j in range(1, KNN):
                            acc = jnp.maximum(acc, rv[_b][i * KNN + j, sl])
                        o_v[i, sl] = jnp.maximum(acc + av[_b][i, sl], 0.0)
                    return c2

                lax.fori_loop(0, _NB, node, 0)
                pltpu.sync_copy(o_v, out_hbm.at[pl.ds(nb, _NB)])

                @pl.when(t + 2 < nch)
                def _():
                    start(b, t + 2)
            return carry

        lax.fori_loop(0, nch // 2, pair, 0)

    return gathermax


def _layer(h, W, b, d_in, d_out, idx_flat):
    wa, wb = W[:d_in], W[d_in:]
    kp = -(-(d_in + 1) // 8) * 8
    dg = max(d_out, 128)
    w_cat = jnp.zeros((kp, 2 * dg), jnp.float32)
    w_cat = w_cat.at[:d_in, :d_out].set(wa - wb)
    w_cat = w_cat.at[d_in, :d_out].set(b)
    w_cat = w_cat.at[:d_in, dg:dg + d_out].set(wb)
    h_aug = jnp.zeros((NP, kp), jnp.float32)
    h_aug = h_aug.at[:, :d_in].set(h)
    h_aug = h_aug.at[:, d_in].set(1.0)
    ac = _matmul(h_aug, w_cat)
    a = ac[:, :dg]
    c = ac[:, dg:]
    out = _make_gathermax(dg)(idx_flat, a, c)
    return out[:, :d_out]


def kernel(x, W0, b0, W1, b1, W2, b2):
    xp = jnp.zeros((NP, KPAD), jnp.float32).at[:N, :3].set(x)
    idx = _knn_idx(xp, xp.T)
    idx_flat = idx.reshape(-1)
    h = _layer(xp[:, :3], W0, b0, 3, 64, idx_flat)
    h = _layer(h, W1, b1, 64, 128, idx_flat)
    h = _layer(h, W2, b2, 128, 256, idx_flat)
    return h[:N]
